# baseline (speedup 1.0000x reference)
import jax
import jax.numpy as jnp
from jax import lax
from jax.experimental import pallas as pl
from jax.experimental.pallas import tpu as pltpu
from jax.experimental.pallas import tpu_sc as plsc

EMBED_DIM = 64
SEQ_LENGTH = 200
BATCH = 4096

NUM_CORES = 2
NUM_SUBCORES = 16
NUM_WORKERS = NUM_CORES * NUM_SUBCORES
BT = BATCH // 128
UNITS = SEQ_LENGTH * BT
UNITS_PER_WORKER = UNITS // NUM_WORKERS
NBUF = 4
GROUPS = UNITS_PER_WORKER // NBUF
LANES = 16


def _body(idx_hbm, word_hbm, pos_hbm, out_hbm, idx_all, pos_vm, gbuf,
          staging, *sems):
    sem_g = sems[0:NBUF]
    sem_o = sems[NBUF:2 * NBUF]
    c = lax.axis_index("c")
    s = lax.axis_index("s")
    wid = s * NUM_CORES + c
    ubase = wid * UNITS_PER_WORKER

    pltpu.sync_copy(idx_hbm.at[pl.ds(ubase, UNITS_PER_WORKER)], idx_all)
    pltpu.sync_copy(pos_hbm, pos_vm)

    def fire_gather(b, r):
        pltpu.async_copy(word_hbm.at[idx_all.at[r]], gbuf.at[b], sem_g[b])

    for b in range(NBUF):
        fire_gather(b, b)

    lane = jnp.arange(LANES, dtype=jnp.int32)

    def group_body(g, carry):
        for b in range(NBUF):
            r = g * NBUF + b
            u = ubase + r
            l = u // BT
            bt = lax.rem(u, BT)
            pltpu.make_async_copy(
                word_hbm.at[idx_all.at[r]], gbuf.at[b], sem_g[b]).wait()

            @pl.when(g > 0)
            def _(b=b):
                for dt in range(8):
                    pltpu.make_async_copy(
                        staging.at[b, dt], out_hbm.at[0, dt, 0],
                        sem_o[b]).wait()

            def d_block(dt2, carry2):
                prow = pos_vm[l, pl.ds(dt2 * LANES, LANES)]
                for dd in range(LANES):
                    d = dt2 * LANES + dd
                    dt = dt2 * 2 + dd // 8
                    di = dd % 8
                    pv = prow[jnp.full((LANES,), dd, dtype=jnp.int32)]
                    dv = lax.broadcast(d, (LANES,))
                    for k in range(8):
                        row = lane + (k * LANES)
                        v = plsc.load_gather(gbuf.at[b], [row, dv])
                        staging[b, dt, di, pl.ds(k * LANES, LANES)] = (
                            v + pv)
                return carry2

            lax.fori_loop(0, EMBED_DIM // LANES, d_block, 0)

            for dt in range(8):
                pltpu.async_copy(
                    staging.at[b, dt], out_hbm.at[l, dt, bt], sem_o[b])
            @pl.when(g < GROUPS - 1)
            def _(b=b, r=r):
                fire_gather(b, r + NBUF)
        return carry

    lax.fori_loop(0, GROUPS, group_body, 0)
    for b in range(NBUF):
        for dt in range(8):
            pltpu.make_async_copy(
                staging.at[b, dt], out_hbm.at[0, dt, 0], sem_o[b]).wait()


@jax.jit
def kernel(inputs, word_table, pos_table):
    idx = (inputs.astype(jnp.int32).T.reshape(UNITS, 128))
    mesh = plsc.VectorSubcoreMesh(
        core_axis_name="c", subcore_axis_name="s")
    run = pl.kernel(
        _body,
        out_type=jax.ShapeDtypeStruct((SEQ_LENGTH, 8, BT, 8, 128),
                                      jnp.float32),
        mesh=mesh,
        scratch_types=[
            pltpu.VMEM((UNITS_PER_WORKER, 128), jnp.int32),
            pltpu.VMEM((SEQ_LENGTH, EMBED_DIM), jnp.float32),
            pltpu.VMEM((NBUF, 128, EMBED_DIM), jnp.float32),
            pltpu.VMEM((NBUF, 8, 8, 128), jnp.float32),
        ] + [pltpu.SemaphoreType.DMA] * (2 * NBUF),
        compiler_params=pltpu.CompilerParams(
            use_tc_tiling_on_sc=False, needs_layout_passes=False),
    )
    out5 = run(idx, word_table, pos_table)
    return out5.transpose(2, 4, 0, 1, 3).reshape(
        BATCH, SEQ_LENGTH, EMBED_DIM)


# speedup vs baseline: 2.0291x; 2.0291x over previous
import jax
import jax.numpy as jnp
from jax import lax
from jax.experimental import pallas as pl
from jax.experimental.pallas import tpu as pltpu
from jax.experimental.pallas import tpu_sc as plsc

EMBED_DIM = 64
SEQ_LENGTH = 200
BATCH = 4096

NUM_CORES = 2
NUM_SUBCORES = 16
NUM_WORKERS = NUM_CORES * NUM_SUBCORES
BT = BATCH // 128
UNITS = SEQ_LENGTH * BT
UNITS_PER_WORKER = UNITS // NUM_WORKERS
NBUF = 2
GROUPS = UNITS_PER_WORKER // NBUF
LANES = 16
SROW = 129


def _body(idx_hbm, word_hbm, pos_hbm, out_hbm, idx_all, pos_sh, gbuf,
          staging, *sems):
    sem_g = sems[0:NBUF]
    sem_p = sems[NBUF:2 * NBUF]
    sem_o = sems[2 * NBUF:3 * NBUF]
    c = lax.axis_index("c")
    s = lax.axis_index("s")
    wid = c * NUM_SUBCORES + s
    ubase = wid * UNITS_PER_WORKER
    lbase = c * (SEQ_LENGTH // NUM_CORES)

    pltpu.sync_copy(idx_hbm.at[pl.ds(ubase, UNITS_PER_WORKER)], idx_all)

    @pl.when(s == 0)
    def _():
        pltpu.sync_copy(
            pos_hbm.at[pl.ds(lbase, SEQ_LENGTH // NUM_CORES)], pos_sh)

    plsc.subcore_barrier()

    def fire_gather(b, r):
        pltpu.async_copy(word_hbm.at[idx_all.at[r]], gbuf.at[b], sem_g[b])

    def fire_prefill(b, l):
        pltpu.async_copy(pos_sh.at[l - lbase], staging.at[b], sem_p[b])

    for b in range(NBUF):
        fire_gather(b, b)
        fire_prefill(b, (ubase + b) // BT)

    lane = jnp.arange(LANES, dtype=jnp.int32)

    def group_body(g, carry):
        for b in range(NBUF):
            r = g * NBUF + b
            u = ubase + r
            l = u // BT
            bt = lax.rem(u, BT)
            pltpu.make_async_copy(
                word_hbm.at[idx_all.at[r]], gbuf.at[b], sem_g[b]).wait()
            pltpu.make_async_copy(
                pos_sh.at[l - lbase], staging.at[b], sem_p[b]).wait()

            def t_body(t, carry2):
                col = lax.broadcast(t, (LANES,))
                for j in range(EMBED_DIM // LANES):
                    v = gbuf[b, t, pl.ds(j * LANES, LANES)]
                    plsc.addupdate_scatter(
                        staging.at[b], [lane + (j * LANES), col], v)
                return carry2

            lax.fori_loop(0, 128, t_body, 0, unroll=4)

            for dt in range(8):
                pltpu.async_copy(
                    staging.at[b, pl.ds(dt * 8, 8), pl.ds(0, 128)],
                    out_hbm.at[l, dt, bt], sem_o[b])
            @pl.when(g < GROUPS - 1)
            def _(b=b, r=r):
                for dt in range(8):
                    pltpu.make_async_copy(
                        staging.at[b, pl.ds(dt * 8, 8), pl.ds(0, 128)],
                        out_hbm.at[0, dt, 0], sem_o[b]).wait()
                fire_gather(b, r + NBUF)
                fire_prefill(b, (ubase + r + NBUF) // BT)
        return carry

    lax.fori_loop(0, GROUPS, group_body, 0)
    for b in range(NBUF):
        for dt in range(8):
            pltpu.make_async_copy(
                staging.at[b, pl.ds(dt * 8, 8), pl.ds(0, 128)],
                out_hbm.at[0, dt, 0], sem_o[b]).wait()


@jax.jit
def kernel(inputs, word_table, pos_table):
    idx = inputs.astype(jnp.int32).T.reshape(UNITS, 128)
    pos_b = jnp.broadcast_to(pos_table[:, :, None],
                             (SEQ_LENGTH, EMBED_DIM, SROW))
    mesh = plsc.VectorSubcoreMesh(
        core_axis_name="c", subcore_axis_name="s")
    run = pl.kernel(
        _body,
        out_type=jax.ShapeDtypeStruct((SEQ_LENGTH, 8, BT, 8, 128),
                                      jnp.float32),
        mesh=mesh,
        scratch_types=[
            pltpu.VMEM((UNITS_PER_WORKER, 128), jnp.int32),
            pltpu.VMEM_SHARED((SEQ_LENGTH // NUM_CORES, EMBED_DIM, SROW),
                              jnp.float32),
            pltpu.VMEM((NBUF, 128, EMBED_DIM), jnp.float32),
            pltpu.VMEM((NBUF, EMBED_DIM, SROW), jnp.float32),
        ] + [pltpu.SemaphoreType.DMA] * (3 * NBUF),
        compiler_params=pltpu.CompilerParams(
            use_tc_tiling_on_sc=False, needs_layout_passes=False),
    )
    out5 = run(idx, word_table, pos_b)
    return out5.transpose(2, 4, 0, 1, 3).reshape(
        BATCH, SEQ_LENGTH, EMBED_DIM)
